# Initial kernel scaffold; baseline (speedup 1.0000x reference)
import jax
import jax.numpy as jnp
from jax.experimental import pallas as pl


def kernel(x, adj_idx, W1, a_src, a_dst, W2, b2, Wb, bb, tcn_w, tcn_b, Wf, bf):
    raise NotImplementedError("write your pallas kernel here")


# speedup vs baseline: 39.6881x; 39.6881x over previous
import functools

import jax
import jax.numpy as jnp
from jax import lax
from jax.experimental import pallas as pl
from jax.experimental.pallas import tpu as pltpu
from jax.experimental.pallas import tpu_sc as plsc

_N = 50000
_E = 800000
_B = 2
_T = 12
_H = 32
_HZ = 12


_BN = 2048
_GRID = (_N + _BN - 1) // _BN


def _pre_body(x_ref, w1_ref, asrc_ref, adst_ref, h_ref, als_ref, ald_ref):
    x2 = x_ref[...].reshape(_B * _BN, _T)
    h2 = jnp.dot(x2, w1_ref[...], preferred_element_type=jnp.float32)
    h3 = h2.reshape(_B, _BN, _H)
    h_ref[...] = h3
    als_ref[...] = jnp.sum(h3 * asrc_ref[...].reshape(1, 1, _H), axis=-1,
                           keepdims=True)
    ald_ref[...] = jnp.sum(h3 * adst_ref[...].reshape(1, 1, _H), axis=-1,
                           keepdims=True)


def _mid_body(agg_ref, w2_ref, b2_ref, w1_ref, asrc_ref, adst_ref,
              h_ref, als_ref, ald_ref):
    a2 = agg_ref[...].reshape(_B * _BN, _H)
    x1 = jnp.maximum(
        jnp.dot(a2, w2_ref[...], preferred_element_type=jnp.float32)
        + b2_ref[...], 0.0)
    h2 = jnp.dot(x1, w1_ref[...], preferred_element_type=jnp.float32)
    h3 = h2.reshape(_B, _BN, _H)
    h_ref[...] = h3
    als_ref[...] = jnp.sum(h3 * asrc_ref[...].reshape(1, 1, _H), axis=-1,
                           keepdims=True)
    ald_ref[...] = jnp.sum(h3 * adst_ref[...].reshape(1, 1, _H), axis=-1,
                           keepdims=True)


def _post_body(agg_ref, w2_ref, b2_ref, wb_ref, bb_ref, tw_ref, tb_ref,
               wf_ref, bf_ref, out_ref):
    a2 = agg_ref[...].reshape(_B * _BN, _H)
    x2 = jnp.maximum(
        jnp.dot(a2, w2_ref[...], preferred_element_type=jnp.float32)
        + b2_ref[...], 0.0)
    so = jnp.dot(x2, wb_ref[...], preferred_element_type=jnp.float32) \
        + bb_ref[...]
    t = so
    for i in range(2):
        xp = jnp.concatenate(
            [jnp.zeros((_B * _BN, 2), jnp.float32), t[:, :_HZ - 2]], axis=1)
        t = jnp.maximum(tw_ref[i, 0] * xp + tw_ref[i, 1] * t + tb_ref[0, i],
                        0.0)
    fu = jnp.concatenate([so, t], axis=1)
    out = jnp.dot(fu, wf_ref[...], preferred_element_type=jnp.float32) \
        + bf_ref[...]
    out_ref[...] = out.reshape(_B, _BN, _HZ)


def _full(shape):
    return pl.BlockSpec(shape, lambda i: tuple(0 for _ in shape))


_pre_call = pl.pallas_call(
    _pre_body,
    grid=(_GRID,),
    in_specs=[
        pl.BlockSpec((_B, _BN, _T), lambda i: (0, i, 0)),
        _full((_T, _H)),
        _full((1, _H)),
        _full((1, _H)),
    ],
    out_specs=[
        pl.BlockSpec((_B, _BN, _H), lambda i: (0, i, 0)),
        pl.BlockSpec((_B, _BN, 1), lambda i: (0, i, 0)),
        pl.BlockSpec((_B, _BN, 1), lambda i: (0, i, 0)),
    ],
    out_shape=[
        jax.ShapeDtypeStruct((_B, _N, _H), jnp.float32),
        jax.ShapeDtypeStruct((_B, _N, 1), jnp.float32),
        jax.ShapeDtypeStruct((_B, _N, 1), jnp.float32),
    ],
)

_mid_call = pl.pallas_call(
    _mid_body,
    grid=(_GRID,),
    in_specs=[
        pl.BlockSpec((_B, _BN, _H), lambda i: (0, i, 0)),
        _full((_H, _T)),
        _full((1, _T)),
        _full((_T, _H)),
        _full((1, _H)),
        _full((1, _H)),
    ],
    out_specs=[
        pl.BlockSpec((_B, _BN, _H), lambda i: (0, i, 0)),
        pl.BlockSpec((_B, _BN, 1), lambda i: (0, i, 0)),
        pl.BlockSpec((_B, _BN, 1), lambda i: (0, i, 0)),
    ],
    out_shape=[
        jax.ShapeDtypeStruct((_B, _N, _H), jnp.float32),
        jax.ShapeDtypeStruct((_B, _N, 1), jnp.float32),
        jax.ShapeDtypeStruct((_B, _N, 1), jnp.float32),
    ],
)

_post_call = pl.pallas_call(
    _post_body,
    grid=(_GRID,),
    in_specs=[
        pl.BlockSpec((_B, _BN, _H), lambda i: (0, i, 0)),
        _full((_H, _T)),
        _full((1, _T)),
        _full((_T, _HZ)),
        _full((1, _HZ)),
        pl.BlockSpec(memory_space=pltpu.SMEM),
        pl.BlockSpec(memory_space=pltpu.SMEM),
        _full((2 * _HZ, _HZ)),
        _full((1, _HZ)),
    ],
    out_specs=pl.BlockSpec((_B, _BN, _HZ), lambda i: (0, i, 0)),
    out_shape=jax.ShapeDtypeStruct((_B, _N, _HZ), jnp.float32),
)


_K = 80
_TPE = _E // 16
_NCH = _TPE // _K
_NSTRIPE = _N // 16
_DZ = 3128


def _sc_body(h_hbm, als_hbm, ald_hbm, src_hbm, dst_hbm, agg_hbm,
             den_sh, agg_sh, als_sh, ald_sh, srcb, dstb, esb, edb, denb,
             attnb, rows, zrow, zflat):
    c = lax.axis_index("c")
    s = lax.axis_index("s")
    bofs = c * _N

    @pl.when(s < 15)
    def _():
        pltpu.sync_copy(als_hbm.at[pl.ds(bofs + s * _DZ, _DZ)],
                        zflat.at[pl.ds(0, _DZ)])
        pltpu.sync_copy(zflat.at[pl.ds(0, _DZ)],
                        als_sh.at[pl.ds(s * _DZ, _DZ)])
        pltpu.sync_copy(ald_hbm.at[pl.ds(bofs + s * _DZ, _DZ)],
                        zflat.at[pl.ds(0, _DZ)])
        pltpu.sync_copy(zflat.at[pl.ds(0, _DZ)],
                        ald_sh.at[pl.ds(s * _DZ, _DZ)])

    _DR = _N - 15 * _DZ

    @pl.when(s == 15)
    def _():
        pltpu.sync_copy(als_hbm.at[pl.ds(bofs + 15 * _DZ, _DR)],
                        zflat.at[pl.ds(0, _DR)])
        pltpu.sync_copy(zflat.at[pl.ds(0, _DR)],
                        als_sh.at[pl.ds(15 * _DZ, _DR)])
        pltpu.sync_copy(ald_hbm.at[pl.ds(bofs + 15 * _DZ, _DR)],
                        zflat.at[pl.ds(0, _DR)])
        pltpu.sync_copy(zflat.at[pl.ds(0, _DR)],
                        ald_sh.at[pl.ds(15 * _DZ, _DR)])

    zv = jnp.zeros((16,), jnp.float32)

    def _zrow_body(i, carry):
        zrow[i, pl.ds(0, 16)] = zv
        zrow[i, pl.ds(16, 16)] = zv
        return carry

    lax.fori_loop(0, 128, _zrow_body, 0)

    def _zflat_body(i, carry):
        zflat[pl.ds(i * 16, 16)] = zv
        return carry

    lax.fori_loop(0, 3136 // 16, _zflat_body, 0)

    @pl.when(s < 15)
    def _():
        pltpu.sync_copy(zflat.at[pl.ds(0, _DZ)],
                        den_sh.at[pl.ds(s * _DZ, _DZ)])

    @pl.when(s == 15)
    def _():
        pltpu.sync_copy(zflat.at[pl.ds(0, _DR)],
                        den_sh.at[pl.ds(15 * _DZ, _DR)])

    for i in range(_NSTRIPE // 125):
        pltpu.sync_copy(zrow.at[pl.ds(0, 125)],
                        agg_sh.at[pl.ds(s * _NSTRIPE + i * 125, 125)])

    plsc.subcore_barrier()

    def _p1_body(j, carry):
        ebase = s * _TPE + j * _K
        pltpu.sync_copy(src_hbm.at[pl.ds(ebase, _K)], srcb)
        pltpu.sync_copy(dst_hbm.at[pl.ds(ebase, _K)], dstb)
        pltpu.sync_copy(als_sh.at[srcb], esb)
        pltpu.sync_copy(ald_sh.at[dstb], edb)
        for g in range(_K // 16):
            e = esb[pl.ds(g * 16, 16)] + edb[pl.ds(g * 16, 16)]
            e = jnp.where(e >= 0.0, e, 0.2 * e)
            attnb[pl.ds(g * 16, 16)] = jnp.exp(e)
        pltpu.sync_copy(attnb, den_sh.at[dstb], add=True)
        return carry

    lax.fori_loop(0, _NCH, _p1_body, 0)
    plsc.subcore_barrier()

    def _p2_body(j, carry):
        ebase = s * _TPE + j * _K
        pltpu.sync_copy(src_hbm.at[pl.ds(ebase, _K)], srcb)
        pltpu.sync_copy(dst_hbm.at[pl.ds(ebase, _K)], dstb)
        pltpu.sync_copy(als_sh.at[srcb], esb)
        pltpu.sync_copy(ald_sh.at[dstb], edb)
        pltpu.sync_copy(den_sh.at[dstb], denb)
        for g in range(_K // 16):
            e = esb[pl.ds(g * 16, 16)] + edb[pl.ds(g * 16, 16)]
            e = jnp.where(e >= 0.0, e, 0.2 * e)
            attnb[pl.ds(g * 16, 16)] = jnp.exp(e) / (denb[pl.ds(g * 16, 16)]
                                                     + 1e-30)
            srcb[pl.ds(g * 16, 16)] = srcb[pl.ds(g * 16, 16)] + bofs
        pltpu.sync_copy(h_hbm.at[srcb], rows)
        for g in range(_K // 16):
            av = attnb[pl.ds(g * 16, 16)]
            for e in range(16):
                a = av[e]
                r = g * 16 + e
                rows[r, pl.ds(0, 16)] = rows[r, pl.ds(0, 16)] * a
                rows[r, pl.ds(16, 16)] = rows[r, pl.ds(16, 16)] * a
        pltpu.sync_copy(rows, agg_sh.at[dstb], add=True)
        return carry

    lax.fori_loop(0, _NCH, _p2_body, 0)
    plsc.subcore_barrier()

    row0 = s * _NSTRIPE
    for i in range(_NSTRIPE // 125):
        pltpu.sync_copy(agg_sh.at[pl.ds(row0 + i * 125, 125)],
                        zrow.at[pl.ds(0, 125)])
        pltpu.sync_copy(zrow.at[pl.ds(0, 125)],
                        agg_hbm.at[pl.ds(bofs + row0 + i * 125, 125)])


_sc_edge = pl.kernel(
    _sc_body,
    out_type=jax.ShapeDtypeStruct((_B * _N, _H), jnp.float32),
    mesh=plsc.VectorSubcoreMesh(core_axis_name="c", subcore_axis_name="s",
                                num_cores=2, num_subcores=16),
    compiler_params=pltpu.CompilerParams(use_tc_tiling_on_sc=False,
                                         needs_layout_passes=False),
    scratch_types=[
        pltpu.VMEM_SHARED((_N,), jnp.float32),
        pltpu.VMEM_SHARED((_N, _H), jnp.float32),
        pltpu.VMEM_SHARED((_N,), jnp.float32),
        pltpu.VMEM_SHARED((_N,), jnp.float32),
        pltpu.VMEM((_K,), jnp.int32),
        pltpu.VMEM((_K,), jnp.int32),
        pltpu.VMEM((_K,), jnp.float32),
        pltpu.VMEM((_K,), jnp.float32),
        pltpu.VMEM((_K,), jnp.float32),
        pltpu.VMEM((_K,), jnp.float32),
        pltpu.VMEM((_K, _H), jnp.float32),
        pltpu.VMEM((128, _H), jnp.float32),
        pltpu.VMEM((3136,), jnp.float32),
    ],
)


def kernel(x, adj_idx, W1, a_src, a_dst, W2, b2, Wb, bb, tcn_w, tcn_b,
           Wf, bf):
    src = adj_idx[0]
    dst = adj_idx[1]
    h, als, ald = _pre_call(x, W1[0], a_src[0].reshape(1, _H),
                            a_dst[0].reshape(1, _H))
    agg0 = _sc_edge(h.reshape(_B * _N, _H), als.reshape(_B * _N),
                    ald.reshape(_B * _N), src, dst)
    h1, als1, ald1 = _mid_call(agg0.reshape(_B, _N, _H), W2[0],
                               b2[0].reshape(1, _T), W1[1],
                               a_src[1].reshape(1, _H),
                               a_dst[1].reshape(1, _H))
    agg1 = _sc_edge(h1.reshape(_B * _N, _H), als1.reshape(_B * _N),
                    ald1.reshape(_B * _N), src, dst)
    out = _post_call(agg1.reshape(_B, _N, _H), W2[1], b2[1].reshape(1, _T),
                     Wb, bb.reshape(1, _HZ), tcn_w, tcn_b.reshape(1, 2),
                     Wf, bf.reshape(1, _HZ))
    return out


# speedup vs baseline: 121.8921x; 3.0713x over previous
import functools

import jax
import jax.numpy as jnp
from jax import lax
from jax.experimental import pallas as pl
from jax.experimental.pallas import tpu as pltpu
from jax.experimental.pallas import tpu_sc as plsc

_N = 50000
_E = 800000
_B = 2
_T = 12
_H = 32
_HZ = 12


_BN = 2048
_GRID = (_N + _BN - 1) // _BN


def _pre_body(x_ref, w1_ref, asrc_ref, adst_ref, h_ref, als_ref, ald_ref):
    x2 = x_ref[...].reshape(_B * _BN, _T)
    h2 = jnp.dot(x2, w1_ref[...], preferred_element_type=jnp.float32)
    h3 = h2.reshape(_B, _BN, _H)
    h_ref[...] = h3
    als_ref[...] = jnp.sum(h3 * asrc_ref[...].reshape(1, 1, _H), axis=-1,
                           keepdims=True)
    ald_ref[...] = jnp.sum(h3 * adst_ref[...].reshape(1, 1, _H), axis=-1,
                           keepdims=True)


def _mid_body(agg_ref, w2_ref, b2_ref, w1_ref, asrc_ref, adst_ref,
              h_ref, als_ref, ald_ref):
    a2 = agg_ref[...].reshape(_B * _BN, _H)
    x1 = jnp.maximum(
        jnp.dot(a2, w2_ref[...], preferred_element_type=jnp.float32)
        + b2_ref[...], 0.0)
    h2 = jnp.dot(x1, w1_ref[...], preferred_element_type=jnp.float32)
    h3 = h2.reshape(_B, _BN, _H)
    h_ref[...] = h3
    als_ref[...] = jnp.sum(h3 * asrc_ref[...].reshape(1, 1, _H), axis=-1,
                           keepdims=True)
    ald_ref[...] = jnp.sum(h3 * adst_ref[...].reshape(1, 1, _H), axis=-1,
                           keepdims=True)


def _post_body(agg_ref, w2_ref, b2_ref, wb_ref, bb_ref, tw_ref, tb_ref,
               wf_ref, bf_ref, out_ref):
    a2 = agg_ref[...].reshape(_B * _BN, _H)
    x2 = jnp.maximum(
        jnp.dot(a2, w2_ref[...], preferred_element_type=jnp.float32)
        + b2_ref[...], 0.0)
    so = jnp.dot(x2, wb_ref[...], preferred_element_type=jnp.float32) \
        + bb_ref[...]
    t = so
    for i in range(2):
        xp = jnp.concatenate(
            [jnp.zeros((_B * _BN, 2), jnp.float32), t[:, :_HZ - 2]], axis=1)
        t = jnp.maximum(tw_ref[i, 0] * xp + tw_ref[i, 1] * t + tb_ref[0, i],
                        0.0)
    fu = jnp.concatenate([so, t], axis=1)
    out = jnp.dot(fu, wf_ref[...], preferred_element_type=jnp.float32) \
        + bf_ref[...]
    out_ref[...] = out.reshape(_B, _BN, _HZ)


def _full(shape):
    return pl.BlockSpec(shape, lambda i: tuple(0 for _ in shape))


_pre_call = pl.pallas_call(
    _pre_body,
    grid=(_GRID,),
    in_specs=[
        pl.BlockSpec((_B, _BN, _T), lambda i: (0, i, 0)),
        _full((_T, _H)),
        _full((1, _H)),
        _full((1, _H)),
    ],
    out_specs=[
        pl.BlockSpec((_B, _BN, _H), lambda i: (0, i, 0)),
        pl.BlockSpec((_B, _BN, 1), lambda i: (0, i, 0)),
        pl.BlockSpec((_B, _BN, 1), lambda i: (0, i, 0)),
    ],
    out_shape=[
        jax.ShapeDtypeStruct((_B, _N, _H), jnp.float32),
        jax.ShapeDtypeStruct((_B, _N, 1), jnp.float32),
        jax.ShapeDtypeStruct((_B, _N, 1), jnp.float32),
    ],
)

_mid_call = pl.pallas_call(
    _mid_body,
    grid=(_GRID,),
    in_specs=[
        pl.BlockSpec((_B, _BN, _H), lambda i: (0, i, 0)),
        _full((_H, _T)),
        _full((1, _T)),
        _full((_T, _H)),
        _full((1, _H)),
        _full((1, _H)),
    ],
    out_specs=[
        pl.BlockSpec((_B, _BN, _H), lambda i: (0, i, 0)),
        pl.BlockSpec((_B, _BN, 1), lambda i: (0, i, 0)),
        pl.BlockSpec((_B, _BN, 1), lambda i: (0, i, 0)),
    ],
    out_shape=[
        jax.ShapeDtypeStruct((_B, _N, _H), jnp.float32),
        jax.ShapeDtypeStruct((_B, _N, 1), jnp.float32),
        jax.ShapeDtypeStruct((_B, _N, 1), jnp.float32),
    ],
)

_post_call = pl.pallas_call(
    _post_body,
    grid=(_GRID,),
    in_specs=[
        pl.BlockSpec((_B, _BN, _H), lambda i: (0, i, 0)),
        _full((_H, _T)),
        _full((1, _T)),
        _full((_T, _HZ)),
        _full((1, _HZ)),
        pl.BlockSpec(memory_space=pltpu.SMEM),
        pl.BlockSpec(memory_space=pltpu.SMEM),
        _full((2 * _HZ, _HZ)),
        _full((1, _HZ)),
    ],
    out_specs=pl.BlockSpec((_B, _BN, _HZ), lambda i: (0, i, 0)),
    out_shape=jax.ShapeDtypeStruct((_B, _N, _HZ), jnp.float32),
)


_K = 400
_TPE = _E // 16
_NCH = _TPE // _K
_NSTRIPE = _N // 16
_DZ = 3128


def _sc_body(h_hbm, als_hbm, ald_hbm, src_hbm, dst_hbm, agg_hbm,
             den_sh, agg_sh, als_sh, ald_sh, srcb, dstb, esb, edb, denb,
             attnb, rows, zrow, zflat, sem0, sem1, sem2, sem3, sem4):
    c = lax.axis_index("c")
    s = lax.axis_index("s")
    bofs = c * _N
    h_v = h_hbm.at[pl.ds(bofs, _N)]

    @pl.when(s == 0)
    def _():
        pltpu.sync_copy(als_hbm.at[pl.ds(bofs, _N)], als_sh)
        pltpu.sync_copy(ald_hbm.at[pl.ds(bofs, _N)], ald_sh)

    zv = jnp.zeros((16,), jnp.float32)

    def _zrow_body(i, carry):
        zrow[i, pl.ds(0, 16)] = zv
        zrow[i, pl.ds(16, 16)] = zv
        return carry

    lax.fori_loop(0, 64, _zrow_body, 0)

    def _zflat_body(i, carry):
        zflat[pl.ds(i * 16, 16)] = zv
        return carry

    lax.fori_loop(0, 3136 // 16, _zflat_body, 0)

    _DR = _N - 15 * _DZ

    @pl.when(s < 15)
    def _():
        pltpu.sync_copy(zflat.at[pl.ds(0, _DZ)],
                        den_sh.at[pl.ds(s * _DZ, _DZ)])

    @pl.when(s == 15)
    def _():
        pltpu.sync_copy(zflat.at[pl.ds(0, _DR)],
                        den_sh.at[pl.ds(15 * _DZ, _DR)])

    for i in range(48):
        pltpu.sync_copy(zrow, agg_sh.at[pl.ds(s * _NSTRIPE + i * 64, 64)])
    pltpu.sync_copy(zrow.at[pl.ds(0, 53)],
                    agg_sh.at[pl.ds(s * _NSTRIPE + 48 * 64, 53)])

    plsc.subcore_barrier()

    def _p1_body(j, carry):
        ebase = s * _TPE + j * _K
        cs = pltpu.async_copy(src_hbm.at[pl.ds(ebase, _K)], srcb, sem0)
        cd = pltpu.async_copy(dst_hbm.at[pl.ds(ebase, _K)], dstb, sem1)
        cs.wait()
        ce = pltpu.async_copy(als_sh.at[srcb], esb, sem0)
        cd.wait()
        cf = pltpu.async_copy(ald_sh.at[dstb], edb, sem1)
        ce.wait()
        cf.wait()

        def _grp(g, carry2):
            e = esb[pl.ds(g * 16, 16)] + edb[pl.ds(g * 16, 16)]
            e = jnp.where(e >= 0.0, e, 0.2 * e)
            attnb[pl.ds(g * 16, 16)] = jnp.exp(e)
            return carry2

        lax.fori_loop(0, _K // 16, _grp, 0)
        pltpu.sync_copy(attnb, den_sh.at[dstb], add=True)
        return carry

    lax.fori_loop(0, _NCH, _p1_body, 0)
    plsc.subcore_barrier()

    def _p2_body(j, carry):
        ebase = s * _TPE + j * _K
        cs = pltpu.async_copy(src_hbm.at[pl.ds(ebase, _K)], srcb, sem0)
        cd = pltpu.async_copy(dst_hbm.at[pl.ds(ebase, _K)], dstb, sem1)
        cs.wait()
        cr = pltpu.async_copy(h_v.at[srcb], rows, sem2)
        ce = pltpu.async_copy(als_sh.at[srcb], esb, sem0)
        cd.wait()
        cf = pltpu.async_copy(ald_sh.at[dstb], edb, sem1)
        cg = pltpu.async_copy(den_sh.at[dstb], denb, sem3)
        ce.wait()
        cf.wait()
        cg.wait()

        def _grp(g, carry2):
            e = esb[pl.ds(g * 16, 16)] + edb[pl.ds(g * 16, 16)]
            e = jnp.where(e >= 0.0, e, 0.2 * e)
            attnb[pl.ds(g * 16, 16)] = jnp.exp(e) / (denb[pl.ds(g * 16, 16)]
                                                     + 1e-30)
            return carry2

        lax.fori_loop(0, _K // 16, _grp, 0)
        cr.wait()

        def _scl(g, carry2):
            av = attnb[pl.ds(g * 16, 16)]
            for e in range(16):
                a = av[e]
                r = g * 16 + e
                rows[r, pl.ds(0, 16)] = rows[r, pl.ds(0, 16)] * a
                rows[r, pl.ds(16, 16)] = rows[r, pl.ds(16, 16)] * a
            return carry2

        lax.fori_loop(0, _K // 16, _scl, 0)
        pltpu.sync_copy(rows, agg_sh.at[dstb], add=True)
        return carry

    lax.fori_loop(0, _NCH, _p2_body, 0)
    plsc.subcore_barrier()

    row0 = s * _NSTRIPE
    pltpu.sync_copy(agg_sh.at[pl.ds(row0, _NSTRIPE)],
                    agg_hbm.at[pl.ds(bofs + row0, _NSTRIPE)])


_sc_edge = pl.kernel(
    _sc_body,
    out_type=jax.ShapeDtypeStruct((_B * _N, _H), jnp.float32),
    mesh=plsc.VectorSubcoreMesh(core_axis_name="c", subcore_axis_name="s",
                                num_cores=2, num_subcores=16),
    compiler_params=pltpu.CompilerParams(use_tc_tiling_on_sc=False,
                                         needs_layout_passes=False),
    scratch_types=[
        pltpu.VMEM_SHARED((_N,), jnp.float32),
        pltpu.VMEM_SHARED((_N, _H), jnp.float32),
        pltpu.VMEM_SHARED((_N,), jnp.float32),
        pltpu.VMEM_SHARED((_N,), jnp.float32),
        pltpu.VMEM((_K,), jnp.int32),
        pltpu.VMEM((_K,), jnp.int32),
        pltpu.VMEM((_K,), jnp.float32),
        pltpu.VMEM((_K,), jnp.float32),
        pltpu.VMEM((_K,), jnp.float32),
        pltpu.VMEM((_K,), jnp.float32),
        pltpu.VMEM((_K, _H), jnp.float32),
        pltpu.VMEM((64, _H), jnp.float32),
        pltpu.VMEM((3136,), jnp.float32),
        pltpu.SemaphoreType.DMA,
        pltpu.SemaphoreType.DMA,
        pltpu.SemaphoreType.DMA,
        pltpu.SemaphoreType.DMA,
        pltpu.SemaphoreType.DMA,
    ],
)


def kernel(x, adj_idx, W1, a_src, a_dst, W2, b2, Wb, bb, tcn_w, tcn_b,
           Wf, bf):
    src = adj_idx[0]
    dst = adj_idx[1]
    h, als, ald = _pre_call(x, W1[0], a_src[0].reshape(1, _H),
                            a_dst[0].reshape(1, _H))
    agg0 = _sc_edge(h.reshape(_B * _N, _H), als.reshape(_B * _N),
                    ald.reshape(_B * _N), src, dst)
    h1, als1, ald1 = _mid_call(agg0.reshape(_B, _N, _H), W2[0],
                               b2[0].reshape(1, _T), W1[1],
                               a_src[1].reshape(1, _H),
                               a_dst[1].reshape(1, _H))
    agg1 = _sc_edge(h1.reshape(_B * _N, _H), als1.reshape(_B * _N),
                    ald1.reshape(_B * _N), src, dst)
    out = _post_call(agg1.reshape(_B, _N, _H), W2[1], b2[1].reshape(1, _T),
                     Wb, bb.reshape(1, _HZ), tcn_w, tcn_b.reshape(1, 2),
                     Wf, bf.reshape(1, _HZ))
    return out


# speedup vs baseline: 133.8093x; 1.0978x over previous
import functools

import jax
import jax.numpy as jnp
from jax import lax
from jax.experimental import pallas as pl
from jax.experimental.pallas import tpu as pltpu
from jax.experimental.pallas import tpu_sc as plsc

_N = 50000
_E = 800000
_B = 2
_T = 12
_H = 32
_HZ = 12


_BN = 2048
_GRID = (_N + _BN - 1) // _BN


def _pre_body(x_ref, w1_ref, asrc_ref, adst_ref, h_ref, als_ref, ald_ref):
    x2 = x_ref[...].reshape(_B * _BN, _T)
    h2 = jnp.dot(x2, w1_ref[...], preferred_element_type=jnp.float32)
    h3 = h2.reshape(_B, _BN, _H)
    h_ref[...] = h3
    als_ref[...] = jnp.sum(h3 * asrc_ref[...].reshape(1, 1, _H), axis=-1,
                           keepdims=True)
    ald_ref[...] = jnp.sum(h3 * adst_ref[...].reshape(1, 1, _H), axis=-1,
                           keepdims=True)


def _mid_body(agg_ref, den_ref, w2_ref, b2_ref, w1_ref, asrc_ref, adst_ref,
              h_ref, als_ref, ald_ref):
    a2 = (agg_ref[...] / (den_ref[...] + 1e-30)).reshape(_B * _BN, _H)
    x1 = jnp.maximum(
        jnp.dot(a2, w2_ref[...], preferred_element_type=jnp.float32)
        + b2_ref[...], 0.0)
    h2 = jnp.dot(x1, w1_ref[...], preferred_element_type=jnp.float32)
    h3 = h2.reshape(_B, _BN, _H)
    h_ref[...] = h3
    als_ref[...] = jnp.sum(h3 * asrc_ref[...].reshape(1, 1, _H), axis=-1,
                           keepdims=True)
    ald_ref[...] = jnp.sum(h3 * adst_ref[...].reshape(1, 1, _H), axis=-1,
                           keepdims=True)


def _post_body(agg_ref, den_ref, w2_ref, b2_ref, wb_ref, bb_ref, tw_ref,
               tb_ref, wf_ref, bf_ref, out_ref):
    a2 = (agg_ref[...] / (den_ref[...] + 1e-30)).reshape(_B * _BN, _H)
    x2 = jnp.maximum(
        jnp.dot(a2, w2_ref[...], preferred_element_type=jnp.float32)
        + b2_ref[...], 0.0)
    so = jnp.dot(x2, wb_ref[...], preferred_element_type=jnp.float32) \
        + bb_ref[...]
    t = so
    for i in range(2):
        xp = jnp.concatenate(
            [jnp.zeros((_B * _BN, 2), jnp.float32), t[:, :_HZ - 2]], axis=1)
        t = jnp.maximum(tw_ref[i, 0] * xp + tw_ref[i, 1] * t + tb_ref[0, i],
                        0.0)
    fu = jnp.concatenate([so, t], axis=1)
    out = jnp.dot(fu, wf_ref[...], preferred_element_type=jnp.float32) \
        + bf_ref[...]
    out_ref[...] = out.reshape(_B, _BN, _HZ)


def _full(shape):
    return pl.BlockSpec(shape, lambda i: tuple(0 for _ in shape))


_pre_call = pl.pallas_call(
    _pre_body,
    grid=(_GRID,),
    in_specs=[
        pl.BlockSpec((_B, _BN, _T), lambda i: (0, i, 0)),
        _full((_T, _H)),
        _full((1, _H)),
        _full((1, _H)),
    ],
    out_specs=[
        pl.BlockSpec((_B, _BN, _H), lambda i: (0, i, 0)),
        pl.BlockSpec((_B, _BN, 1), lambda i: (0, i, 0)),
        pl.BlockSpec((_B, _BN, 1), lambda i: (0, i, 0)),
    ],
    out_shape=[
        jax.ShapeDtypeStruct((_B, _N, _H), jnp.float32),
        jax.ShapeDtypeStruct((_B, _N, 1), jnp.float32),
        jax.ShapeDtypeStruct((_B, _N, 1), jnp.float32),
    ],
)

_mid_call = pl.pallas_call(
    _mid_body,
    grid=(_GRID,),
    in_specs=[
        pl.BlockSpec((_B, _BN, _H), lambda i: (0, i, 0)),
        pl.BlockSpec((_B, _BN, 1), lambda i: (0, i, 0)),
        _full((_H, _T)),
        _full((1, _T)),
        _full((_T, _H)),
        _full((1, _H)),
        _full((1, _H)),
    ],
    out_specs=[
        pl.BlockSpec((_B, _BN, _H), lambda i: (0, i, 0)),
        pl.BlockSpec((_B, _BN, 1), lambda i: (0, i, 0)),
        pl.BlockSpec((_B, _BN, 1), lambda i: (0, i, 0)),
    ],
    out_shape=[
        jax.ShapeDtypeStruct((_B, _N, _H), jnp.float32),
        jax.ShapeDtypeStruct((_B, _N, 1), jnp.float32),
        jax.ShapeDtypeStruct((_B, _N, 1), jnp.float32),
    ],
)

_post_call = pl.pallas_call(
    _post_body,
    grid=(_GRID,),
    in_specs=[
        pl.BlockSpec((_B, _BN, _H), lambda i: (0, i, 0)),
        pl.BlockSpec((_B, _BN, 1), lambda i: (0, i, 0)),
        _full((_H, _T)),
        _full((1, _T)),
        _full((_T, _HZ)),
        _full((1, _HZ)),
        pl.BlockSpec(memory_space=pltpu.SMEM),
        pl.BlockSpec(memory_space=pltpu.SMEM),
        _full((2 * _HZ, _HZ)),
        _full((1, _HZ)),
    ],
    out_specs=pl.BlockSpec((_B, _BN, _HZ), lambda i: (0, i, 0)),
    out_shape=jax.ShapeDtypeStruct((_B, _N, _HZ), jnp.float32),
)


_K = 400
_TPE = _E // 16
_NCH = _TPE // _K
_NSTRIPE = _N // 16
_DZ = 3128


def _sc_body(h_hbm, als_hbm, ald_hbm, src_hbm, dst_hbm, agg_hbm, den_hbm,
             den_sh, agg_sh, als_sh, ald_sh, srcb, dstb, esb, edb,
             attnb, rows, zrow, zflat, sem0, sem1, sem2, sem3, sem4):
    c = lax.axis_index("c")
    s = lax.axis_index("s")
    bofs = c * _N
    h_v = h_hbm.at[pl.ds(bofs, _N)]

    @pl.when(s == 0)
    def _():
        pltpu.sync_copy(als_hbm.at[pl.ds(bofs, _N)], als_sh)
        pltpu.sync_copy(ald_hbm.at[pl.ds(bofs, _N)], ald_sh)

    zv = jnp.zeros((16,), jnp.float32)

    def _zrow_body(i, carry):
        zrow[i, pl.ds(0, 16)] = zv
        zrow[i, pl.ds(16, 16)] = zv
        return carry

    lax.fori_loop(0, 64, _zrow_body, 0)

    def _zflat_body(i, carry):
        zflat[pl.ds(i * 16, 16)] = zv
        return carry

    lax.fori_loop(0, 3136 // 16, _zflat_body, 0)

    _DR = _N - 15 * _DZ

    @pl.when(s < 15)
    def _():
        pltpu.sync_copy(zflat.at[pl.ds(0, _DZ)],
                        den_sh.at[pl.ds(s * _DZ, _DZ)])

    @pl.when(s == 15)
    def _():
        pltpu.sync_copy(zflat.at[pl.ds(0, _DR)],
                        den_sh.at[pl.ds(15 * _DZ, _DR)])

    for i in range(48):
        pltpu.sync_copy(zrow, agg_sh.at[pl.ds(s * _NSTRIPE + i * 64, 64)])
    pltpu.sync_copy(zrow.at[pl.ds(0, 53)],
                    agg_sh.at[pl.ds(s * _NSTRIPE + 48 * 64, 53)])

    plsc.subcore_barrier()

    def _pp_body(j, carry):
        ebase = s * _TPE + j * _K
        cs = pltpu.async_copy(src_hbm.at[pl.ds(ebase, _K)], srcb, sem0)
        cd = pltpu.async_copy(dst_hbm.at[pl.ds(ebase, _K)], dstb, sem1)
        cs.wait()
        cr = pltpu.async_copy(h_v.at[srcb], rows, sem2)
        ce = pltpu.async_copy(als_sh.at[srcb], esb, sem0)
        cd.wait()
        cf = pltpu.async_copy(ald_sh.at[dstb], edb, sem1)
        ce.wait()
        cf.wait()

        def _grp(g, carry2):
            e = esb[pl.ds(g * 16, 16)] + edb[pl.ds(g * 16, 16)]
            e = jnp.where(e >= 0.0, e, 0.2 * e)
            attnb[pl.ds(g * 16, 16)] = jnp.exp(e)
            return carry2

        lax.fori_loop(0, _K // 16, _grp, 0)
        cden = pltpu.async_copy(attnb, den_sh.at[dstb], sem3, add=True)
        cr.wait()

        def _scl(g, carry2):
            av = attnb[pl.ds(g * 16, 16)]
            for e in range(16):
                a = av[e]
                r = g * 16 + e
                rows[r, pl.ds(0, 16)] = rows[r, pl.ds(0, 16)] * a
                rows[r, pl.ds(16, 16)] = rows[r, pl.ds(16, 16)] * a
            return carry2

        lax.fori_loop(0, _K // 16, _scl, 0)
        cden.wait()
        pltpu.sync_copy(rows, agg_sh.at[dstb], add=True)
        return carry

    lax.fori_loop(0, _NCH, _pp_body, 0)
    plsc.subcore_barrier()

    row0 = s * _NSTRIPE
    pltpu.sync_copy(agg_sh.at[pl.ds(row0, _NSTRIPE)],
                    agg_hbm.at[pl.ds(bofs + row0, _NSTRIPE)])

    @pl.when(s < 15)
    def _():
        pltpu.sync_copy(den_sh.at[pl.ds(s * _DZ, _DZ)],
                        den_hbm.at[pl.ds(bofs + s * _DZ, _DZ)])

    @pl.when(s == 15)
    def _():
        pltpu.sync_copy(den_sh.at[pl.ds(15 * _DZ, _DR)],
                        den_hbm.at[pl.ds(bofs + 15 * _DZ, _DR)])


_sc_edge = pl.kernel(
    _sc_body,
    out_type=(jax.ShapeDtypeStruct((_B * _N, _H), jnp.float32),
              jax.ShapeDtypeStruct((_B * _N,), jnp.float32)),
    mesh=plsc.VectorSubcoreMesh(core_axis_name="c", subcore_axis_name="s",
                                num_cores=2, num_subcores=16),
    compiler_params=pltpu.CompilerParams(use_tc_tiling_on_sc=False,
                                         needs_layout_passes=False),
    scratch_types=[
        pltpu.VMEM_SHARED((_N,), jnp.float32),
        pltpu.VMEM_SHARED((_N, _H), jnp.float32),
        pltpu.VMEM_SHARED((_N,), jnp.float32),
        pltpu.VMEM_SHARED((_N,), jnp.float32),
        pltpu.VMEM((_K,), jnp.int32),
        pltpu.VMEM((_K,), jnp.int32),
        pltpu.VMEM((_K,), jnp.float32),
        pltpu.VMEM((_K,), jnp.float32),
        pltpu.VMEM((_K,), jnp.float32),
        pltpu.VMEM((_K, _H), jnp.float32),
        pltpu.VMEM((64, _H), jnp.float32),
        pltpu.VMEM((3136,), jnp.float32),
        pltpu.SemaphoreType.DMA,
        pltpu.SemaphoreType.DMA,
        pltpu.SemaphoreType.DMA,
        pltpu.SemaphoreType.DMA,
        pltpu.SemaphoreType.DMA,
    ],
)


def kernel(x, adj_idx, W1, a_src, a_dst, W2, b2, Wb, bb, tcn_w, tcn_b,
           Wf, bf):
    src = adj_idx[0]
    dst = adj_idx[1]
    h, als, ald = _pre_call(x, W1[0], a_src[0].reshape(1, _H),
                            a_dst[0].reshape(1, _H))
    agg0, den0 = _sc_edge(h.reshape(_B * _N, _H), als.reshape(_B * _N),
                          ald.reshape(_B * _N), src, dst)
    h1, als1, ald1 = _mid_call(agg0.reshape(_B, _N, _H),
                               den0.reshape(_B, _N, 1), W2[0],
                               b2[0].reshape(1, _T), W1[1],
                               a_src[1].reshape(1, _H),
                               a_dst[1].reshape(1, _H))
    agg1, den1 = _sc_edge(h1.reshape(_B * _N, _H), als1.reshape(_B * _N),
                          ald1.reshape(_B * _N), src, dst)
    out = _post_call(agg1.reshape(_B, _N, _H), den1.reshape(_B, _N, 1),
                     W2[1], b2[1].reshape(1, _T),
                     Wb, bb.reshape(1, _HZ), tcn_w, tcn_b.reshape(1, 2),
                     Wf, bf.reshape(1, _HZ))
    return out


# speedup vs baseline: 160.4933x; 1.1994x over previous
import functools

import jax
import jax.numpy as jnp
from jax import lax
from jax.experimental import pallas as pl
from jax.experimental.pallas import tpu as pltpu
from jax.experimental.pallas import tpu_sc as plsc

_N = 50000
_E = 800000
_B = 2
_T = 12
_H = 32
_HZ = 12


_BN = 2048
_GRID = (_N + _BN - 1) // _BN


def _pre_body(x_ref, w1_ref, asrc_ref, adst_ref, h_ref, als_ref, ald_ref):
    x2 = x_ref[...].reshape(_B * _BN, _T)
    h2 = jnp.dot(x2, w1_ref[...], preferred_element_type=jnp.float32)
    h3 = h2.reshape(_B, _BN, _H)
    h_ref[...] = h3
    als_ref[...] = jnp.sum(h3 * asrc_ref[...].reshape(1, 1, _H), axis=-1)
    ald_ref[...] = jnp.sum(h3 * adst_ref[...].reshape(1, 1, _H), axis=-1)


def _mid_body(agg_ref, den_ref, w2_ref, b2_ref, w1_ref, asrc_ref, adst_ref,
              h_ref, als_ref, ald_ref):
    a3 = agg_ref[...] / (den_ref[...][:, :, None] + 1e-30)
    a2 = a3.reshape(_B * _BN, _H)
    x1 = jnp.maximum(
        jnp.dot(a2, w2_ref[...], preferred_element_type=jnp.float32)
        + b2_ref[...], 0.0)
    h2 = jnp.dot(x1, w1_ref[...], preferred_element_type=jnp.float32)
    h3 = h2.reshape(_B, _BN, _H)
    h_ref[...] = h3
    als_ref[...] = jnp.sum(h3 * asrc_ref[...].reshape(1, 1, _H), axis=-1)
    ald_ref[...] = jnp.sum(h3 * adst_ref[...].reshape(1, 1, _H), axis=-1)


def _post_body(agg_ref, den_ref, w2_ref, b2_ref, wb_ref, bb_ref, tw_ref,
               tb_ref, wf_ref, bf_ref, out_ref):
    a3 = agg_ref[...] / (den_ref[...][:, :, None] + 1e-30)
    a2 = a3.reshape(_B * _BN, _H)
    x2 = jnp.maximum(
        jnp.dot(a2, w2_ref[...], preferred_element_type=jnp.float32)
        + b2_ref[...], 0.0)
    so = jnp.dot(x2, wb_ref[...], preferred_element_type=jnp.float32) \
        + bb_ref[...]
    t = so
    for i in range(2):
        xp = jnp.concatenate(
            [jnp.zeros((_B * _BN, 2), jnp.float32), t[:, :_HZ - 2]], axis=1)
        t = jnp.maximum(tw_ref[i, 0] * xp + tw_ref[i, 1] * t + tb_ref[0, i],
                        0.0)
    fu = jnp.concatenate([so, t], axis=1)
    out = jnp.dot(fu, wf_ref[...], preferred_element_type=jnp.float32) \
        + bf_ref[...]
    out_ref[...] = out.reshape(_B, _BN, _HZ)


def _full(shape):
    return pl.BlockSpec(shape, lambda i: tuple(0 for _ in shape))


_pre_call = pl.pallas_call(
    _pre_body,
    grid=(_GRID,),
    in_specs=[
        pl.BlockSpec((_B, _BN, _T), lambda i: (0, i, 0)),
        _full((_T, _H)),
        _full((1, _H)),
        _full((1, _H)),
    ],
    out_specs=[
        pl.BlockSpec((_B, _BN, _H), lambda i: (0, i, 0)),
        pl.BlockSpec((_B, _BN), lambda i: (0, i)),
        pl.BlockSpec((_B, _BN), lambda i: (0, i)),
    ],
    out_shape=[
        jax.ShapeDtypeStruct((_B, _N, _H), jnp.float32),
        jax.ShapeDtypeStruct((_B, _N), jnp.float32),
        jax.ShapeDtypeStruct((_B, _N), jnp.float32),
    ],
)

_mid_call = pl.pallas_call(
    _mid_body,
    grid=(_GRID,),
    in_specs=[
        pl.BlockSpec((_B, _BN, _H), lambda i: (0, i, 0)),
        pl.BlockSpec((_B, _BN), lambda i: (0, i)),
        _full((_H, _T)),
        _full((1, _T)),
        _full((_T, _H)),
        _full((1, _H)),
        _full((1, _H)),
    ],
    out_specs=[
        pl.BlockSpec((_B, _BN, _H), lambda i: (0, i, 0)),
        pl.BlockSpec((_B, _BN), lambda i: (0, i)),
        pl.BlockSpec((_B, _BN), lambda i: (0, i)),
    ],
    out_shape=[
        jax.ShapeDtypeStruct((_B, _N, _H), jnp.float32),
        jax.ShapeDtypeStruct((_B, _N), jnp.float32),
        jax.ShapeDtypeStruct((_B, _N), jnp.float32),
    ],
)

_post_call = pl.pallas_call(
    _post_body,
    grid=(_GRID,),
    in_specs=[
        pl.BlockSpec((_B, _BN, _H), lambda i: (0, i, 0)),
        pl.BlockSpec((_B, _BN), lambda i: (0, i)),
        _full((_H, _T)),
        _full((1, _T)),
        _full((_T, _HZ)),
        _full((1, _HZ)),
        pl.BlockSpec(memory_space=pltpu.SMEM),
        pl.BlockSpec(memory_space=pltpu.SMEM),
        _full((2 * _HZ, _HZ)),
        _full((1, _HZ)),
    ],
    out_specs=pl.BlockSpec((_B, _BN, _HZ), lambda i: (0, i, 0)),
    out_shape=jax.ShapeDtypeStruct((_B, _N, _HZ), jnp.float32),
)


_K = 400
_TPE = _E // 16
_NCH = _TPE // _K
_NSTRIPE = _N // 16
_DZ = 3128


def _sc_body(h_hbm, als_hbm, ald_hbm, adj_hbm, z1_hbm, z2_hbm,
             agg_hbm, den_hbm,
             den_sh, agg_sh, als_sh, ald_sh, srcb, dstb, esb, edb,
             attnb, rows, sem0, sem1, sem2, sem3, sem4):
    c = lax.axis_index("c")
    s = lax.axis_index("s")
    h_v = h_hbm.at[c]

    @pl.when(s == 0)
    def _():
        pltpu.sync_copy(als_hbm.at[c], als_sh)
        pltpu.sync_copy(ald_hbm.at[c], ald_sh)

    _DR = _N - 15 * _DZ

    @pl.when(s < 15)
    def _():
        pltpu.sync_copy(z1_hbm.at[pl.ds(0, _DZ)],
                        den_sh.at[pl.ds(s * _DZ, _DZ)])

    @pl.when(s == 15)
    def _():
        pltpu.sync_copy(z1_hbm.at[pl.ds(0, _DR)],
                        den_sh.at[pl.ds(15 * _DZ, _DR)])

    pltpu.sync_copy(z2_hbm.at[pl.ds(0, _NSTRIPE)],
                    agg_sh.at[pl.ds(s * _NSTRIPE, _NSTRIPE)])

    plsc.subcore_barrier()

    def _pp_body(j, carry):
        ebase = s * _TPE + j * _K
        cs = pltpu.async_copy(adj_hbm.at[0, pl.ds(ebase, _K)], srcb, sem0)
        cd = pltpu.async_copy(adj_hbm.at[1, pl.ds(ebase, _K)], dstb, sem1)
        cs.wait()
        cr = pltpu.async_copy(h_v.at[srcb], rows, sem2)
        ce = pltpu.async_copy(als_sh.at[srcb], esb, sem0)
        cd.wait()
        cf = pltpu.async_copy(ald_sh.at[dstb], edb, sem1)
        ce.wait()
        cf.wait()

        def _grp(g, carry2):
            e = esb[pl.ds(g * 16, 16)] + edb[pl.ds(g * 16, 16)]
            e = jnp.where(e >= 0.0, e, 0.2 * e)
            attnb[pl.ds(g * 16, 16)] = jnp.exp(e)
            return carry2

        lax.fori_loop(0, _K // 16, _grp, 0)
        cden = pltpu.async_copy(attnb, den_sh.at[dstb], sem3, add=True)
        cr.wait()

        def _scl(g, carry2):
            av = attnb[pl.ds(g * 16, 16)]
            for e in range(16):
                a = av[e]
                r = g * 16 + e
                rows[r, pl.ds(0, 16)] = rows[r, pl.ds(0, 16)] * a
                rows[r, pl.ds(16, 16)] = rows[r, pl.ds(16, 16)] * a
            return carry2

        lax.fori_loop(0, _K // 16, _scl, 0)
        cden.wait()
        pltpu.sync_copy(rows, agg_sh.at[dstb], add=True)
        return carry

    lax.fori_loop(0, _NCH, _pp_body, 0)
    plsc.subcore_barrier()

    row0 = s * _NSTRIPE
    pltpu.sync_copy(agg_sh.at[pl.ds(row0, _NSTRIPE)],
                    agg_hbm.at[c, pl.ds(row0, _NSTRIPE)])

    @pl.when(s < 15)
    def _():
        pltpu.sync_copy(den_sh.at[pl.ds(s * _DZ, _DZ)],
                        den_hbm.at[c, pl.ds(s * _DZ, _DZ)])

    @pl.when(s == 15)
    def _():
        pltpu.sync_copy(den_sh.at[pl.ds(15 * _DZ, _DR)],
                        den_hbm.at[c, pl.ds(15 * _DZ, _DR)])


_sc_edge = pl.kernel(
    _sc_body,
    out_type=(jax.ShapeDtypeStruct((_B, _N, _H), jnp.float32),
              jax.ShapeDtypeStruct((_B, _N), jnp.float32)),
    mesh=plsc.VectorSubcoreMesh(core_axis_name="c", subcore_axis_name="s",
                                num_cores=2, num_subcores=16),
    compiler_params=pltpu.CompilerParams(use_tc_tiling_on_sc=False,
                                         needs_layout_passes=False),
    scratch_types=[
        pltpu.VMEM_SHARED((_N,), jnp.float32),
        pltpu.VMEM_SHARED((_N, _H), jnp.float32),
        pltpu.VMEM_SHARED((_N,), jnp.float32),
        pltpu.VMEM_SHARED((_N,), jnp.float32),
        pltpu.VMEM((_K,), jnp.int32),
        pltpu.VMEM((_K,), jnp.int32),
        pltpu.VMEM((_K,), jnp.float32),
        pltpu.VMEM((_K,), jnp.float32),
        pltpu.VMEM((_K,), jnp.float32),
        pltpu.VMEM((_K, _H), jnp.float32),
        pltpu.SemaphoreType.DMA,
        pltpu.SemaphoreType.DMA,
        pltpu.SemaphoreType.DMA,
        pltpu.SemaphoreType.DMA,
        pltpu.SemaphoreType.DMA,
    ],
)


def kernel(x, adj_idx, W1, a_src, a_dst, W2, b2, Wb, bb, tcn_w, tcn_b,
           Wf, bf):
    z1 = jnp.zeros((3136,), jnp.float32)
    z2 = jnp.zeros((3136, _H), jnp.float32)
    h, als, ald = _pre_call(x, W1[0], a_src[0].reshape(1, _H),
                            a_dst[0].reshape(1, _H))
    agg0, den0 = _sc_edge(h, als, ald, adj_idx, z1, z2)
    h1, als1, ald1 = _mid_call(agg0, den0, W2[0],
                               b2[0].reshape(1, _T), W1[1],
                               a_src[1].reshape(1, _H),
                               a_dst[1].reshape(1, _H))
    agg1, den1 = _sc_edge(h1, als1, ald1, adj_idx, z1, z2)
    out = _post_call(agg1, den1, W2[1], b2[1].reshape(1, _T),
                     Wb, bb.reshape(1, _HZ), tcn_w, tcn_b.reshape(1, 2),
                     Wf, bf.reshape(1, _HZ))
    return out


# speedup vs baseline: 166.8727x; 1.0397x over previous
import functools

import jax
import jax.numpy as jnp
from jax import lax
from jax.experimental import pallas as pl
from jax.experimental.pallas import tpu as pltpu
from jax.experimental.pallas import tpu_sc as plsc

_N = 50000
_E = 800000
_B = 2
_T = 12
_H = 32
_HZ = 12


_BN = 2048
_GRID = (_N + _BN - 1) // _BN


def _pre_body(x_ref, w1_ref, asrc_ref, adst_ref, h_ref, als_ref, ald_ref):
    x2 = x_ref[...].reshape(_B * _BN, _T)
    h2 = jnp.dot(x2, w1_ref[...], preferred_element_type=jnp.float32)
    h3 = h2.reshape(_B, _BN, _H)
    h_ref[...] = h3
    als_ref[...] = jnp.sum(h3 * asrc_ref[...].reshape(1, 1, _H), axis=-1)
    ald_ref[...] = jnp.sum(h3 * adst_ref[...].reshape(1, 1, _H), axis=-1)


def _mid_body(agg_ref, den_ref, w2_ref, b2_ref, w1_ref, asrc_ref, adst_ref,
              h_ref, als_ref, ald_ref):
    a3 = agg_ref[...] / (den_ref[...][:, :, None] + 1e-30)
    a2 = a3.reshape(_B * _BN, _H)
    x1 = jnp.maximum(
        jnp.dot(a2, w2_ref[...], preferred_element_type=jnp.float32)
        + b2_ref[...], 0.0)
    h2 = jnp.dot(x1, w1_ref[...], preferred_element_type=jnp.float32)
    h3 = h2.reshape(_B, _BN, _H)
    h_ref[...] = h3
    als_ref[...] = jnp.sum(h3 * asrc_ref[...].reshape(1, 1, _H), axis=-1)
    ald_ref[...] = jnp.sum(h3 * adst_ref[...].reshape(1, 1, _H), axis=-1)


def _post_body(agg_ref, den_ref, w2_ref, b2_ref, wb_ref, bb_ref, tw_ref,
               tb_ref, wf_ref, bf_ref, out_ref):
    a3 = agg_ref[...] / (den_ref[...][:, :, None] + 1e-30)
    a2 = a3.reshape(_B * _BN, _H)
    x2 = jnp.maximum(
        jnp.dot(a2, w2_ref[...], preferred_element_type=jnp.float32)
        + b2_ref[...], 0.0)
    so = jnp.dot(x2, wb_ref[...], preferred_element_type=jnp.float32) \
        + bb_ref[...]
    t = so
    for i in range(2):
        xp = jnp.concatenate(
            [jnp.zeros((_B * _BN, 2), jnp.float32), t[:, :_HZ - 2]], axis=1)
        t = jnp.maximum(tw_ref[i, 0] * xp + tw_ref[i, 1] * t + tb_ref[0, i],
                        0.0)
    fu = jnp.concatenate([so, t], axis=1)
    out = jnp.dot(fu, wf_ref[...], preferred_element_type=jnp.float32) \
        + bf_ref[...]
    out_ref[...] = out.reshape(_B, _BN, _HZ)


def _full(shape):
    return pl.BlockSpec(shape, lambda i: tuple(0 for _ in shape))


_pre_call = pl.pallas_call(
    _pre_body,
    grid=(_GRID,),
    in_specs=[
        pl.BlockSpec((_B, _BN, _T), lambda i: (0, i, 0)),
        _full((_T, _H)),
        _full((1, _H)),
        _full((1, _H)),
    ],
    out_specs=[
        pl.BlockSpec((_B, _BN, _H), lambda i: (0, i, 0)),
        pl.BlockSpec((_B, _BN), lambda i: (0, i)),
        pl.BlockSpec((_B, _BN), lambda i: (0, i)),
    ],
    out_shape=[
        jax.ShapeDtypeStruct((_B, _N, _H), jnp.float32),
        jax.ShapeDtypeStruct((_B, _N), jnp.float32),
        jax.ShapeDtypeStruct((_B, _N), jnp.float32),
    ],
)

_mid_call = pl.pallas_call(
    _mid_body,
    grid=(_GRID,),
    in_specs=[
        pl.BlockSpec((_B, _BN, _H), lambda i: (0, i, 0)),
        pl.BlockSpec((_B, _BN), lambda i: (0, i)),
        _full((_H, _T)),
        _full((1, _T)),
        _full((_T, _H)),
        _full((1, _H)),
        _full((1, _H)),
    ],
    out_specs=[
        pl.BlockSpec((_B, _BN, _H), lambda i: (0, i, 0)),
        pl.BlockSpec((_B, _BN), lambda i: (0, i)),
        pl.BlockSpec((_B, _BN), lambda i: (0, i)),
    ],
    out_shape=[
        jax.ShapeDtypeStruct((_B, _N, _H), jnp.float32),
        jax.ShapeDtypeStruct((_B, _N), jnp.float32),
        jax.ShapeDtypeStruct((_B, _N), jnp.float32),
    ],
)

_post_call = pl.pallas_call(
    _post_body,
    grid=(_GRID,),
    in_specs=[
        pl.BlockSpec((_B, _BN, _H), lambda i: (0, i, 0)),
        pl.BlockSpec((_B, _BN), lambda i: (0, i)),
        _full((_H, _T)),
        _full((1, _T)),
        _full((_T, _HZ)),
        _full((1, _HZ)),
        pl.BlockSpec(memory_space=pltpu.SMEM),
        pl.BlockSpec(memory_space=pltpu.SMEM),
        _full((2 * _HZ, _HZ)),
        _full((1, _HZ)),
    ],
    out_specs=pl.BlockSpec((_B, _BN, _HZ), lambda i: (0, i, 0)),
    out_shape=jax.ShapeDtypeStruct((_B, _N, _HZ), jnp.float32),
)


_K = 400
_TPE = _E // 16
_NCH = _TPE // _K
_NSTRIPE = _N // 16
_DZ = 3128


_KA = 208
_KB = 192


def _sc_body(h_hbm, als_hbm, ald_hbm, adj_hbm, z1_hbm, z2_hbm,
             agg_hbm, den_hbm,
             den_sh, agg_sh, als_sh, ald_sh,
             srcba, srcbb, dstba, dstbb, esba, esbb, edba, edbb,
             attnba, attnbb, rowsa, rowsb,
             sem0, sem1, sem2, sem3, sem4, sem5, sem6, sem7):
    c = lax.axis_index("c")
    s = lax.axis_index("s")
    h_v = h_hbm.at[c]

    @pl.when(s == 0)
    def _():
        pltpu.sync_copy(als_hbm.at[c], als_sh)
        pltpu.sync_copy(ald_hbm.at[c], ald_sh)

    _DR = _N - 15 * _DZ

    @pl.when(s < 15)
    def _():
        pltpu.sync_copy(z1_hbm.at[pl.ds(0, _DZ)],
                        den_sh.at[pl.ds(s * _DZ, _DZ)])

    @pl.when(s == 15)
    def _():
        pltpu.sync_copy(z1_hbm.at[pl.ds(0, _DR)],
                        den_sh.at[pl.ds(15 * _DZ, _DR)])

    pltpu.sync_copy(z2_hbm.at[pl.ds(0, _NSTRIPE)],
                    agg_sh.at[pl.ds(s * _NSTRIPE, _NSTRIPE)])

    plsc.subcore_barrier()

    def _mk_grp(esb, edb, attnb):
        def _grp(g, carry2):
            e = esb[pl.ds(g * 16, 16)] + edb[pl.ds(g * 16, 16)]
            e = jnp.where(e >= 0.0, e, 0.2 * e)
            attnb[pl.ds(g * 16, 16)] = jnp.exp(e)
            return carry2
        return _grp

    def _mk_scl(attnb, rows):
        def _scl(g, carry2):
            av = attnb[pl.ds(g * 16, 16)]
            for e in range(16):
                a = av[e]
                r = g * 16 + e
                rows[r, pl.ds(0, 16)] = rows[r, pl.ds(0, 16)] * a
                rows[r, pl.ds(16, 16)] = rows[r, pl.ds(16, 16)] * a
            return carry2
        return _scl

    def _pp_body(j, carry):
        ebase = s * _TPE + j * _K
        ca0 = pltpu.async_copy(adj_hbm.at[0, pl.ds(ebase, _KA)], srcba, sem0)
        cb0 = pltpu.async_copy(adj_hbm.at[1, pl.ds(ebase, _KA)], dstba, sem1)
        ca1 = pltpu.async_copy(adj_hbm.at[0, pl.ds(ebase + _KA, _KB)],
                               srcbb, sem2)
        cb1 = pltpu.async_copy(adj_hbm.at[1, pl.ds(ebase + _KA, _KB)],
                               dstbb, sem3)
        ca0.wait()
        cra = pltpu.async_copy(h_v.at[srcba], rowsa, sem4)
        cea = pltpu.async_copy(als_sh.at[srcba], esba, sem0)
        cb0.wait()
        cfa = pltpu.async_copy(ald_sh.at[dstba], edba, sem1)
        ca1.wait()
        crb = pltpu.async_copy(h_v.at[srcbb], rowsb, sem5)
        ceb = pltpu.async_copy(als_sh.at[srcbb], esbb, sem2)
        cb1.wait()
        cfb = pltpu.async_copy(ald_sh.at[dstbb], edbb, sem3)
        cea.wait()
        cfa.wait()
        lax.fori_loop(0, _KA // 16, _mk_grp(esba, edba, attnba), 0)
        cdena = pltpu.async_copy(attnba, den_sh.at[dstba], sem6, add=True)
        cra.wait()
        lax.fori_loop(0, _KA // 16, _mk_scl(attnba, rowsa), 0)
        csa = pltpu.async_copy(rowsa, agg_sh.at[dstba], sem7, add=True)
        ceb.wait()
        cfb.wait()
        lax.fori_loop(0, _KB // 16, _mk_grp(esbb, edbb, attnbb), 0)
        cdenb = pltpu.async_copy(attnbb, den_sh.at[dstbb], sem6, add=True)
        crb.wait()
        lax.fori_loop(0, _KB // 16, _mk_scl(attnbb, rowsb), 0)
        csb = pltpu.async_copy(rowsb, agg_sh.at[dstbb], sem7, add=True)
        cdena.wait()
        csa.wait()
        cdenb.wait()
        csb.wait()
        return carry

    lax.fori_loop(0, _NCH, _pp_body, 0)
    plsc.subcore_barrier()

    row0 = s * _NSTRIPE
    pltpu.sync_copy(agg_sh.at[pl.ds(row0, _NSTRIPE)],
                    agg_hbm.at[c, pl.ds(row0, _NSTRIPE)])

    @pl.when(s < 15)
    def _():
        pltpu.sync_copy(den_sh.at[pl.ds(s * _DZ, _DZ)],
                        den_hbm.at[c, pl.ds(s * _DZ, _DZ)])

    @pl.when(s == 15)
    def _():
        pltpu.sync_copy(den_sh.at[pl.ds(15 * _DZ, _DR)],
                        den_hbm.at[c, pl.ds(15 * _DZ, _DR)])


_sc_edge = pl.kernel(
    _sc_body,
    out_type=(jax.ShapeDtypeStruct((_B, _N, _H), jnp.float32),
              jax.ShapeDtypeStruct((_B, _N), jnp.float32)),
    mesh=plsc.VectorSubcoreMesh(core_axis_name="c", subcore_axis_name="s",
                                num_cores=2, num_subcores=16),
    compiler_params=pltpu.CompilerParams(use_tc_tiling_on_sc=False,
                                         needs_layout_passes=False),
    scratch_types=[
        pltpu.VMEM_SHARED((_N,), jnp.float32),
        pltpu.VMEM_SHARED((_N, _H), jnp.float32),
        pltpu.VMEM_SHARED((_N,), jnp.float32),
        pltpu.VMEM_SHARED((_N,), jnp.float32),
        pltpu.VMEM((_KA,), jnp.int32),
        pltpu.VMEM((_KB,), jnp.int32),
        pltpu.VMEM((_KA,), jnp.int32),
        pltpu.VMEM((_KB,), jnp.int32),
        pltpu.VMEM((_KA,), jnp.float32),
        pltpu.VMEM((_KB,), jnp.float32),
        pltpu.VMEM((_KA,), jnp.float32),
        pltpu.VMEM((_KB,), jnp.float32),
        pltpu.VMEM((_KA,), jnp.float32),
        pltpu.VMEM((_KB,), jnp.float32),
        pltpu.VMEM((_KA, _H), jnp.float32),
        pltpu.VMEM((_KB, _H), jnp.float32),
        pltpu.SemaphoreType.DMA,
        pltpu.SemaphoreType.DMA,
        pltpu.SemaphoreType.DMA,
        pltpu.SemaphoreType.DMA,
        pltpu.SemaphoreType.DMA,
        pltpu.SemaphoreType.DMA,
        pltpu.SemaphoreType.DMA,
        pltpu.SemaphoreType.DMA,
    ],
)


def kernel(x, adj_idx, W1, a_src, a_dst, W2, b2, Wb, bb, tcn_w, tcn_b,
           Wf, bf):
    z1 = jnp.zeros((3136,), jnp.float32)
    z2 = jnp.zeros((3136, _H), jnp.float32)
    h, als, ald = _pre_call(x, W1[0], a_src[0].reshape(1, _H),
                            a_dst[0].reshape(1, _H))
    agg0, den0 = _sc_edge(h, als, ald, adj_idx, z1, z2)
    h1, als1, ald1 = _mid_call(agg0, den0, W2[0],
                               b2[0].reshape(1, _T), W1[1],
                               a_src[1].reshape(1, _H),
                               a_dst[1].reshape(1, _H))
    agg1, den1 = _sc_edge(h1, als1, ald1, adj_idx, z1, z2)
    out = _post_call(agg1, den1, W2[1], b2[1].reshape(1, _T),
                     Wb, bb.reshape(1, _HZ), tcn_w, tcn_b.reshape(1, 2),
                     Wf, bf.reshape(1, _HZ))
    return out
